# baseline (speedup 1.0000x reference)
import functools

import jax
import jax.numpy as jnp
from jax import lax
from jax.experimental import pallas as pl
from jax.experimental.pallas import tpu as pltpu
from jax.experimental.pallas import tpu_sc as plsc

NC = 2
NS = 16
CH = 128
NIDX = 4
NROW = 2
DH = 80


def _sc_segment_sum(xs, edges, zeros, n_pad, cpt):
    rps = n_pad // NS

    mesh = plsc.VectorSubcoreMesh(core_axis_name="c", subcore_axis_name="s")

    @functools.partial(
        pl.kernel,
        out_type=jax.ShapeDtypeStruct((NC * n_pad, DH), jnp.float32),
        mesh=mesh,
        scratch_types=[
            pltpu.VMEM((NIDX, 2, CH), jnp.int32),
            pltpu.VMEM((NROW, CH, DH), jnp.float32),
            pltpu.VMEM_SHARED((n_pad, DH), jnp.float32),
            pltpu.VMEM_SHARED((n_pad, DH), jnp.float32),
            pltpu.SemaphoreType.DMA,
            pltpu.SemaphoreType.DMA,
            pltpu.SemaphoreType.DMA,
            pltpu.SemaphoreType.DMA,
            pltpu.SemaphoreType.DMA,
            pltpu.SemaphoreType.DMA,
        ],
        compiler_params=pltpu.CompilerParams(use_tc_tiling_on_sc=False),
    )
    def sc_kernel(xs_hbm, edges_hbm, zeros_hbm, out_hbm,
                  idx_v, rows_v, tab_sh, acc_sh, i0, i1, i2, i3, g0, g1):
        isem = (i0, i1, i2, i3)
        gsem = (g0, g1)
        cid = lax.axis_index("c")
        sid = lax.axis_index("s")
        base = sid * cpt
        rows = pl.ds(sid * rps, rps)
        crows = pl.ds(cid * n_pad + sid * rps, rps)

        def idx_load(c, slot):
            return pltpu.make_async_copy(
                edges_hbm.at[pl.ds(2 * (base + c), 2)],
                idx_v.at[slot], isem[slot])

        def gather(c_slot, r_slot):
            return pltpu.make_async_copy(tab_sh.at[idx_v.at[c_slot, 0]],
                                         rows_v.at[r_slot], gsem[r_slot])

        pltpu.sync_copy(xs_hbm.at[crows], tab_sh.at[rows])
        pltpu.sync_copy(zeros_hbm.at[rows], acc_sh.at[rows])
        plsc.subcore_barrier()

        for s in range(NIDX):
            idx_load(s, s).start()
        for bn in range(NROW):
            idx_load(bn, bn).wait()
            gather(bn, bn).start()

        def step(j, bn, refill):
            gather(bn, bn % NROW).wait()
            pltpu.sync_copy(rows_v.at[bn % NROW],
                            acc_sh.at[idx_v.at[bn, 1]], add=True)
            if refill:
                idx_load(j + bn + NIDX, bn).start()
            if bn < NROW or refill:
                idx_load(j + bn + NROW, (bn + NROW) % NIDX).wait()
                gather((bn + NROW) % NIDX, bn % NROW).start()

        @pl.loop(0, cpt - NIDX, step=NIDX)
        def _(j):
            for bn in range(NIDX):
                step(j, bn, refill=True)

        for bn in range(NIDX):
            step(cpt - NIDX, bn, refill=False)

        plsc.subcore_barrier()
        pltpu.sync_copy(acc_sh.at[rows], out_hbm.at[crows])

    return sc_kernel(xs, edges, zeros)


def _tc_combine(x, s0, s1, W, b2, n, d, blk):
    h = d // 2

    def body(x_ref, s0_ref, s1_ref, w_ref, b_ref, o_ref):
        deg = s0_ref[:, h:h + 1]
        xw = jnp.dot(x_ref[...], w_ref[:d, :], preferred_element_type=jnp.float32)
        sw = (jnp.dot(s0_ref[:, :h], w_ref[d:d + h, :],
                      preferred_element_type=jnp.float32)
              + jnp.dot(s1_ref[:, :h], w_ref[d + h:, :],
                        preferred_element_type=jnp.float32))
        o_ref[...] = deg * (xw + b_ref[...]) + sw

    return pl.pallas_call(
        body,
        grid=(n // blk,),
        in_specs=[
            pl.BlockSpec((blk, d), lambda i: (i, 0)),
            pl.BlockSpec((blk, DH), lambda i: (i, 0)),
            pl.BlockSpec((blk, DH), lambda i: (i, 0)),
            pl.BlockSpec((2 * d, d), lambda i: (0, 0)),
            pl.BlockSpec((1, d), lambda i: (0, 0)),
        ],
        out_specs=pl.BlockSpec((blk, d), lambda i: (i, 0)),
        out_shape=jax.ShapeDtypeStruct((n, d), jnp.float32),
    )(x, s0, s1, W, b2)


def kernel(x, edge_index, W, b):
    n, d = x.shape
    e = edge_index.shape[1]
    h = d // 2

    cpt = -(-e // (CH * NS))
    cpt = -(-cpt // NIDX) * NIDX
    e_pad = NS * cpt * CH
    n_pad = -(-n // (NS * 8)) * (NS * 8)

    xs = jnp.zeros((NC * n_pad, DH), jnp.float32)
    xs = xs.at[:n, :h].set(x[:, :h]).at[:n, h].set(1.0)
    xs = xs.at[n_pad:n_pad + n, :h].set(x[:, h:])
    src = jnp.concatenate(
        [edge_index[0], jnp.zeros((e_pad - e,), jnp.int32)]).reshape(-1, CH)
    dst = jnp.concatenate(
        [edge_index[1], jnp.full((e_pad - e,), n, jnp.int32)]).reshape(-1, CH)
    edges = jnp.stack([src, dst], axis=1).reshape(-1, CH)
    zeros = jnp.zeros((n_pad, DH), jnp.float32)

    parts = _sc_segment_sum(xs, edges, zeros, n_pad, cpt)

    blk = 1000 if n % 1000 == 0 else 8
    return _tc_combine(x, parts[:n], parts[n_pad:n_pad + n], W,
                       b.reshape(1, d), n, d, blk)


# speedup vs baseline: 1.1200x; 1.1200x over previous
import functools

import jax
import jax.numpy as jnp
from jax import lax
from jax.experimental import pallas as pl
from jax.experimental.pallas import tpu as pltpu
from jax.experimental.pallas import tpu_sc as plsc

NC = 2
NS = 16
CH = 128
NIDX = 8
NROW = 4
DH = 72


def _sc_segment_sum(xs, edges, zeros, n_pad, cpt):
    rps = n_pad // NS

    mesh = plsc.VectorSubcoreMesh(core_axis_name="c", subcore_axis_name="s")

    @functools.partial(
        pl.kernel,
        out_type=jax.ShapeDtypeStruct((NC * n_pad, 128), jnp.float32),
        mesh=mesh,
        scratch_types=[
            pltpu.VMEM((NIDX, 2, CH), jnp.int32),
            pltpu.VMEM((NROW, CH, DH), jnp.float32),
            pltpu.VMEM_SHARED((n_pad, DH), jnp.float32),
            pltpu.VMEM_SHARED((n_pad, DH), jnp.float32),
            pltpu.SemaphoreType.DMA,
            pltpu.SemaphoreType.DMA,
            pltpu.SemaphoreType.DMA,
            pltpu.SemaphoreType.DMA,
            pltpu.SemaphoreType.DMA,
            pltpu.SemaphoreType.DMA,
            pltpu.SemaphoreType.DMA,
            pltpu.SemaphoreType.DMA,
            pltpu.SemaphoreType.DMA,
            pltpu.SemaphoreType.DMA,
            pltpu.SemaphoreType.DMA,
            pltpu.SemaphoreType.DMA,
        ],
        compiler_params=pltpu.CompilerParams(use_tc_tiling_on_sc=False),
    )
    def sc_kernel(xs_hbm, edges_hbm, zeros_hbm, out_hbm,
                  idx_v, rows_v, tab_sh, acc_sh,
                  i0, i1, i2, i3, i4, i5, i6, i7, g0, g1, g2, g3):
        isem = (i0, i1, i2, i3, i4, i5, i6, i7)
        gsem = (g0, g1, g2, g3)
        cid = lax.axis_index("c")
        sid = lax.axis_index("s")
        base = sid * cpt
        rows = pl.ds(sid * rps, rps)
        crows = pl.ds(cid * n_pad + sid * rps, rps)

        def idx_load(c, slot):
            return pltpu.make_async_copy(
                edges_hbm.at[pl.ds(2 * (base + c), 2)],
                idx_v.at[slot], isem[slot])

        def gather(c_slot, r_slot):
            return pltpu.make_async_copy(tab_sh.at[idx_v.at[c_slot, 0]],
                                         rows_v.at[r_slot], gsem[r_slot])

        cols = pl.ds(0, DH)
        pltpu.sync_copy(xs_hbm.at[crows, cols], tab_sh.at[rows])
        pltpu.sync_copy(zeros_hbm.at[rows, cols], acc_sh.at[rows])
        plsc.subcore_barrier()

        for s in range(NIDX):
            idx_load(s, s).start()
        for bn in range(NROW):
            idx_load(bn, bn).wait()
            gather(bn, bn).start()

        def step(j, bn, refill):
            gather(bn, bn % NROW).wait()
            pltpu.sync_copy(rows_v.at[bn % NROW],
                            acc_sh.at[idx_v.at[bn, 1]], add=True)
            if refill:
                idx_load(j + bn + NIDX, bn).start()
            if bn < NROW or refill:
                idx_load(j + bn + NROW, (bn + NROW) % NIDX).wait()
                gather((bn + NROW) % NIDX, bn % NROW).start()

        @pl.loop(0, cpt - NIDX, step=NIDX)
        def _(j):
            for bn in range(NIDX):
                step(j, bn, refill=True)

        for bn in range(NIDX):
            step(cpt - NIDX, bn, refill=False)

        plsc.subcore_barrier()
        pltpu.sync_copy(acc_sh.at[rows], out_hbm.at[crows, cols])

    return sc_kernel(xs, edges, zeros)


def _tc_combine(x, s0, s1, W, b2, n, d, blk):
    h = d // 2

    def body(x_ref, s0_ref, s1_ref, w_ref, b_ref, o_ref):
        deg = s0_ref[:, h:h + 1]
        xw = jnp.dot(x_ref[...], w_ref[:d, :], preferred_element_type=jnp.float32)
        sw = (jnp.dot(s0_ref[:, :h], w_ref[d:d + h, :],
                      preferred_element_type=jnp.float32)
              + jnp.dot(s1_ref[:, :h], w_ref[d + h:, :],
                        preferred_element_type=jnp.float32))
        o_ref[...] = deg * (xw + b_ref[...]) + sw

    return pl.pallas_call(
        body,
        grid=(n // blk,),
        in_specs=[
            pl.BlockSpec((blk, d), lambda i: (i, 0)),
            pl.BlockSpec((blk, d), lambda i: (i, 0)),
            pl.BlockSpec((blk, d), lambda i: (i, 0)),
            pl.BlockSpec((2 * d, d), lambda i: (0, 0)),
            pl.BlockSpec((1, d), lambda i: (0, 0)),
        ],
        out_specs=pl.BlockSpec((blk, d), lambda i: (i, 0)),
        out_shape=jax.ShapeDtypeStruct((n, d), jnp.float32),
    )(x, s0, s1, W, b2)


def kernel(x, edge_index, W, b):
    n, d = x.shape
    e = edge_index.shape[1]
    h = d // 2

    cpt = -(-e // (CH * NS))
    cpt = -(-cpt // NIDX) * NIDX
    e_pad = NS * cpt * CH
    n_pad = -(-n // (NS * 8)) * (NS * 8)

    xs = jnp.zeros((NC * n_pad, d), jnp.float32)
    xs = xs.at[:n, :h].set(x[:, :h]).at[:n, h].set(1.0)
    xs = xs.at[n_pad:n_pad + n, :h].set(x[:, h:])
    src = jnp.concatenate(
        [edge_index[0], jnp.zeros((e_pad - e,), jnp.int32)]).reshape(-1, CH)
    dst = jnp.concatenate(
        [edge_index[1], jnp.full((e_pad - e,), n, jnp.int32)]).reshape(-1, CH)
    edges = jnp.stack([src, dst], axis=1).reshape(-1, CH)
    zeros = jnp.zeros((n_pad, d), jnp.float32)

    parts = _sc_segment_sum(xs, edges, zeros, n_pad, cpt)

    blk = 1000 if n % 1000 == 0 else 8
    return _tc_combine(x, parts[:n], parts[n_pad:n_pad + n], W,
                       b.reshape(1, d), n, d, blk)
